# Initial kernel scaffold; baseline (speedup 1.0000x reference)
import jax
import jax.numpy as jnp
from jax import lax
from jax.experimental import pallas as pl
from jax.experimental.pallas import tpu as pltpu
from jax.experimental.pallas import tpu_sc as plsc

NC = 2
NS = 16
NW = NC * NS
LANES = 16
CHUNK = 128


def _sc_edge_pass(n_pad, d, k_chunks, with_gather):
    mesh = plsc.VectorSubcoreMesh(core_axis_name="c", subcore_axis_name="s")
    rps = n_pad // NS

    def body(*refs):
        if with_gather:
            hs_hbm, src_hbm, dst_hbm, out_hbm, src_v, dst_v, rows_v, acc = refs
        else:
            dst_hbm, out_hbm, dst_v, rows_v, acc = refs
        c = lax.axis_index("c")
        s = lax.axis_index("s")
        wid = s * NC + c

        @pl.loop(0, CHUNK)
        def _(i):
            for j in range(d // LANES):
                rows_v.at[i, pl.ds(j * LANES, LANES)][...] = jnp.zeros(
                    (LANES,), jnp.float32)

        @pl.loop(0, rps // CHUNK)
        def _(t):
            pltpu.sync_copy(rows_v, acc.at[pl.ds(s * rps + t * CHUNK, CHUNK)])

        if not with_gather:
            @pl.loop(0, CHUNK)
            def _(i):
                for j in range(d // LANES):
                    rows_v.at[i, pl.ds(j * LANES, LANES)][...] = jnp.full(
                        (LANES,), 1.0, jnp.float32)

        pltpu.sync_copy(dst_hbm.at[wid], dst_v)
        if with_gather:
            pltpu.sync_copy(src_hbm.at[wid], src_v)
        plsc.subcore_barrier()

        if with_gather:
            @pl.loop(0, k_chunks)
            def _(k):
                pltpu.sync_copy(hs_hbm.at[src_v.at[k]], rows_v)
                pltpu.sync_copy(rows_v, acc.at[dst_v.at[k]], add=True)
        else:
            @pl.loop(0, k_chunks)
            def _(k):
                pltpu.sync_copy(rows_v, acc.at[dst_v.at[k]], add=True)

        plsc.subcore_barrier()
        pltpu.sync_copy(acc.at[pl.ds(s * rps, rps)],
                        out_hbm.at[c, pl.ds(s * rps, rps)])

    scratch = [
        pltpu.VMEM((k_chunks, CHUNK), jnp.int32),
        pltpu.VMEM((CHUNK, d), jnp.float32),
        pltpu.VMEM_SHARED((n_pad, d), jnp.float32),
    ]
    if with_gather:
        scratch.insert(0, pltpu.VMEM((k_chunks, CHUNK), jnp.int32))

    return pl.kernel(
        body,
        out_type=jax.ShapeDtypeStruct((NC, n_pad, d), jnp.float32),
        mesh=mesh,
        scratch_types=scratch,
    )


def _tc_matmul(x_pad, w):
    def mm(x_ref, w_ref, o_ref):
        o_ref[...] = jnp.dot(x_ref[...], w_ref[...],
                             preferred_element_type=jnp.float32)
    out = jax.ShapeDtypeStruct((x_pad.shape[0], w.shape[1]), jnp.float32)
    return pl.pallas_call(mm, out_shape=out)(x_pad, w)


def _tc_scale1(degp, h1):
    n_pad, d = h1.shape

    def body(degp_ref, h1_ref, dis_ref, h1s_ref):
        deg = degp_ref[0, :, 0:1] + degp_ref[1, :, 0:1] + 1.0
        dis = lax.rsqrt(deg)
        dis_ref[...] = dis
        h1s_ref[...] = h1_ref[...] * dis

    outs = (jax.ShapeDtypeStruct((n_pad, 1), jnp.float32),
            jax.ShapeDtypeStruct((n_pad, d), jnp.float32))
    return pl.pallas_call(body, out_shape=outs)(degp, h1)


def _tc_layer2_prep(p1, h1s, dis, b1r, w2p, n_real):
    n_pad = h1s.shape[0]
    d2 = w2p.shape[1]

    def body(p_ref, h1s_ref, dis_ref, b1_ref, w2_ref, o_ref):
        dis = dis_ref[...]
        agg = (p_ref[0] + p_ref[1] + h1s_ref[...]) * dis + b1_ref[...]
        out1 = jnp.maximum(agg, 0.0)
        h2 = jnp.dot(out1, w2_ref[...], preferred_element_type=jnp.float32)
        mask = lax.broadcasted_iota(jnp.int32, (n_pad, 1), 0) < n_real
        o_ref[...] = jnp.where(mask, h2 * dis, 0.0)

    out = jax.ShapeDtypeStruct((n_pad, d2), jnp.float32)
    return pl.pallas_call(body, out_shape=out)(p1, h1s, dis, b1r, w2p)


def _tc_final(p2, h2s, dis, b2r, n_real, d_out):
    def body(p_ref, h2s_ref, dis_ref, b2_ref, o_ref):
        z = (p_ref[0, :n_real, :d_out] + p_ref[1, :n_real, :d_out]
             + h2s_ref[:n_real, :d_out]) * dis_ref[:n_real] + b2_ref[...]
        m = jnp.max(z, axis=1, keepdims=True)
        zm = z - m
        lse = jnp.log(jnp.sum(jnp.exp(zm), axis=1, keepdims=True))
        o_ref[...] = zm - lse

    out = jax.ShapeDtypeStruct((n_real, d_out), jnp.float32)
    return pl.pallas_call(body, out_shape=out)(p2, h2s, dis, b2r)


def kernel(x, edge_index, W1, b1, W2, b2):
    n, d_in = x.shape
    d_hid = W1.shape[1]
    d_out = W2.shape[1]
    e = edge_index.shape[1]

    n_pad = -(-(n + 1) // (NS * CHUNK)) * (NS * CHUNK)
    d_out_pad = -(-d_out // LANES) * LANES
    k_chunks = -(-e // (NW * CHUNK))
    e_pad = k_chunks * NW * CHUNK

    fill = jnp.full((e_pad - e,), n, jnp.int32)
    src3 = jnp.concatenate([edge_index[0], fill]).reshape(NW, k_chunks, CHUNK)
    dst3 = jnp.concatenate([edge_index[1], fill]).reshape(NW, k_chunks, CHUNK)

    x_pad = jnp.pad(x, ((0, n_pad - n), (0, 0)))
    w2p = jnp.pad(W2, ((0, 0), (0, d_out_pad - d_out)))
    b1r = b1.reshape(1, d_hid)
    b2r = b2.reshape(1, d_out)

    degp = _sc_edge_pass(n_pad, LANES, k_chunks, with_gather=False)(dst3)
    h1 = _tc_matmul(x_pad, W1)
    dis, h1s = _tc_scale1(degp, h1)
    p1 = _sc_edge_pass(n_pad, d_hid, k_chunks, with_gather=True)(h1s, src3, dst3)
    h2s = _tc_layer2_prep(p1, h1s, dis, b1r, w2p, n)
    p2 = _sc_edge_pass(n_pad, d_out_pad, k_chunks, with_gather=True)(h2s, src3, dst3)
    return _tc_final(p2, h2s, dis, b2r, n, d_out)


# speedup vs baseline: 25.6069x; 25.6069x over previous
import jax
import jax.numpy as jnp
from jax import lax
from jax.experimental import pallas as pl
from jax.experimental.pallas import tpu as pltpu
from jax.experimental.pallas import tpu_sc as plsc

NC = 2
NS = 16
NW = NC * NS
LANES = 16
CHUNK = 128


def _sc_edge_pass(n_pad, d, k_chunks, with_gather):
    mesh = plsc.VectorSubcoreMesh(core_axis_name="c", subcore_axis_name="s")
    rps = n_pad // NS

    def body(*refs):
        if with_gather:
            hs_hbm, src_hbm, dst_hbm, out_hbm, src_v, dst_v, rows_v, acc = refs
        else:
            dst_hbm, out_hbm, dst_v, rows_v, acc = refs
        c = lax.axis_index("c")
        s = lax.axis_index("s")
        wid = s * NC + c

        @pl.loop(0, CHUNK)
        def _(i):
            for j in range(d // LANES):
                rows_v.at[i, pl.ds(j * LANES, LANES)][...] = jnp.zeros(
                    (LANES,), jnp.float32)

        @pl.loop(0, rps // CHUNK)
        def _(t):
            pltpu.sync_copy(rows_v, acc.at[pl.ds(s * rps + t * CHUNK, CHUNK)])

        if not with_gather:
            @pl.loop(0, CHUNK)
            def _(i):
                for j in range(d // LANES):
                    rows_v.at[i, pl.ds(j * LANES, LANES)][...] = jnp.full(
                        (LANES,), 1.0, jnp.float32)

        pltpu.sync_copy(dst_hbm.at[wid], dst_v)
        if with_gather:
            pltpu.sync_copy(src_hbm.at[wid], src_v)
        plsc.subcore_barrier()

        if with_gather:
            @pl.loop(0, k_chunks)
            def _(k):
                pltpu.sync_copy(hs_hbm.at[src_v.at[k]], rows_v)
                pltpu.sync_copy(rows_v, acc.at[dst_v.at[k]], add=True)
        else:
            @pl.loop(0, k_chunks)
            def _(k):
                pltpu.sync_copy(rows_v, acc.at[dst_v.at[k]], add=True)

        plsc.subcore_barrier()
        pltpu.sync_copy(acc.at[pl.ds(s * rps, rps)],
                        out_hbm.at[c, pl.ds(s * rps, rps)])

    scratch = [
        pltpu.VMEM((k_chunks, CHUNK), jnp.int32),
        pltpu.VMEM((CHUNK, d), jnp.float32),
        pltpu.VMEM_SHARED((n_pad, d), jnp.float32),
    ]
    if with_gather:
        scratch.insert(0, pltpu.VMEM((k_chunks, CHUNK), jnp.int32))

    return pl.kernel(
        body,
        out_type=jax.ShapeDtypeStruct((NC, n_pad, d), jnp.float32),
        mesh=mesh,
        scratch_types=scratch,
        compiler_params=pltpu.CompilerParams(use_tc_tiling_on_sc=False),
    )


def _tc_matmul(x_pad, w):
    def mm(x_ref, w_ref, o_ref):
        o_ref[...] = jnp.dot(x_ref[...], w_ref[...],
                             preferred_element_type=jnp.float32)
    out = jax.ShapeDtypeStruct((x_pad.shape[0], w.shape[1]), jnp.float32)
    return pl.pallas_call(mm, out_shape=out)(x_pad, w)


def _tc_scale1(degp, h1):
    n_pad, d = h1.shape

    def body(degp_ref, h1_ref, dis_ref, h1s_ref):
        deg = degp_ref[0, :, 0:1] + degp_ref[1, :, 0:1] + 1.0
        dis = lax.rsqrt(deg)
        dis_ref[...] = dis
        h1s_ref[...] = h1_ref[...] * dis

    outs = (jax.ShapeDtypeStruct((n_pad, 1), jnp.float32),
            jax.ShapeDtypeStruct((n_pad, d), jnp.float32))
    return pl.pallas_call(body, out_shape=outs)(degp, h1)


def _tc_layer2_prep(p1, h1s, dis, b1r, w2p, n_real):
    n_pad = h1s.shape[0]
    d2 = w2p.shape[1]

    def body(p_ref, h1s_ref, dis_ref, b1_ref, w2_ref, o_ref):
        dis = dis_ref[...]
        agg = (p_ref[0] + p_ref[1] + h1s_ref[...]) * dis + b1_ref[...]
        out1 = jnp.maximum(agg, 0.0)
        h2 = jnp.dot(out1, w2_ref[...], preferred_element_type=jnp.float32)
        mask = lax.broadcasted_iota(jnp.int32, (n_pad, 1), 0) < n_real
        o_ref[...] = jnp.where(mask, h2 * dis, 0.0)

    out = jax.ShapeDtypeStruct((n_pad, d2), jnp.float32)
    return pl.pallas_call(body, out_shape=out)(p1, h1s, dis, b1r, w2p)


def _tc_final(p2, h2s, dis, b2r, n_real, d_out):
    def body(p_ref, h2s_ref, dis_ref, b2_ref, o_ref):
        z = (p_ref[0, :n_real, :d_out] + p_ref[1, :n_real, :d_out]
             + h2s_ref[:n_real, :d_out]) * dis_ref[:n_real] + b2_ref[...]
        m = jnp.max(z, axis=1, keepdims=True)
        zm = z - m
        lse = jnp.log(jnp.sum(jnp.exp(zm), axis=1, keepdims=True))
        o_ref[...] = zm - lse

    out = jax.ShapeDtypeStruct((n_real, d_out), jnp.float32)
    return pl.pallas_call(body, out_shape=out)(p2, h2s, dis, b2r)


def kernel(x, edge_index, W1, b1, W2, b2):
    n, d_in = x.shape
    d_hid = W1.shape[1]
    d_out = W2.shape[1]
    e = edge_index.shape[1]

    n_pad = -(-(n + 1) // (NS * CHUNK)) * (NS * CHUNK)
    d_out_pad = -(-d_out // LANES) * LANES
    k_chunks = -(-e // (NW * CHUNK))
    e_pad = k_chunks * NW * CHUNK

    fill = jnp.full((e_pad - e,), n, jnp.int32)
    src3 = jnp.concatenate([edge_index[0], fill]).reshape(NW, k_chunks, CHUNK)
    dst3 = jnp.concatenate([edge_index[1], fill]).reshape(NW, k_chunks, CHUNK)

    x_pad = jnp.pad(x, ((0, n_pad - n), (0, 0)))
    w2p = jnp.pad(W2, ((0, 0), (0, d_out_pad - d_out)))
    b1r = b1.reshape(1, d_hid)
    b2r = b2.reshape(1, d_out)

    degp = _sc_edge_pass(n_pad, LANES, k_chunks, with_gather=False)(dst3)
    h1 = _tc_matmul(x_pad, W1)
    dis, h1s = _tc_scale1(degp, h1)
    p1 = _sc_edge_pass(n_pad, d_hid, k_chunks, with_gather=True)(h1s, src3, dst3)
    h2s = _tc_layer2_prep(p1, h1s, dis, b1r, w2p, n)
    p2 = _sc_edge_pass(n_pad, d_out_pad, k_chunks, with_gather=True)(h2s, src3, dst3)
    return _tc_final(p2, h2s, dis, b2r, n, d_out)


# speedup vs baseline: 27.2271x; 1.0633x over previous
import jax
import jax.numpy as jnp
from jax import lax
from jax.experimental import pallas as pl
from jax.experimental.pallas import tpu as pltpu
from jax.experimental.pallas import tpu_sc as plsc

NC = 2
NS = 16
NW = NC * NS
LANES = 16
CHUNK = 128
NBUF = 8


def _sc_edge_pass(n_pad, d, k_chunks, with_gather):
    mesh = plsc.VectorSubcoreMesh(core_axis_name="c", subcore_axis_name="s")
    rps = n_pad // NS

    def body(*refs):
        if with_gather:
            (hs_hbm, src_hbm, dst_hbm, out_hbm,
             src_v, dst_v, rows_v, acc), sems = refs[:8], refs[8:]
        else:
            dst_hbm, out_hbm, dst_v, rows_v, acc = refs
        c = lax.axis_index("c")
        s = lax.axis_index("s")
        wid = s * NC + c
        zbuf = rows_v.at[0] if with_gather else rows_v

        @pl.loop(0, CHUNK)
        def _(i):
            for j in range(d // LANES):
                zbuf.at[i, pl.ds(j * LANES, LANES)][...] = jnp.zeros(
                    (LANES,), jnp.float32)

        @pl.loop(0, rps // CHUNK)
        def _(t):
            pltpu.sync_copy(zbuf, acc.at[pl.ds(s * rps + t * CHUNK, CHUNK)])

        if not with_gather:
            @pl.loop(0, CHUNK)
            def _(i):
                for j in range(d // LANES):
                    rows_v.at[i, pl.ds(j * LANES, LANES)][...] = jnp.full(
                        (LANES,), 1.0, jnp.float32)

        pltpu.sync_copy(dst_hbm.at[wid], dst_v)
        if with_gather:
            pltpu.sync_copy(src_hbm.at[wid], src_v)
        plsc.subcore_barrier()

        if with_gather:
            for b in range(NBUF):
                pltpu.async_copy(hs_hbm.at[src_v.at[b]], rows_v.at[b], sems[b])

            @pl.loop(0, k_chunks, step=NBUF)
            def _(k):
                for b in range(NBUF):
                    pltpu.make_async_copy(
                        hs_hbm.at[src_v.at[b]], rows_v.at[b], sems[b]).wait()
                    pltpu.sync_copy(rows_v.at[b], acc.at[dst_v.at[k + b]],
                                    add=True)
                    nxt = k + b + NBUF

                    @pl.when(nxt < k_chunks)
                    def _():
                        pltpu.async_copy(
                            hs_hbm.at[src_v.at[nxt]], rows_v.at[b], sems[b])
        else:
            @pl.loop(0, k_chunks)
            def _(k):
                pltpu.sync_copy(rows_v, acc.at[dst_v.at[k]], add=True)

        plsc.subcore_barrier()
        pltpu.sync_copy(acc.at[pl.ds(s * rps, rps)],
                        out_hbm.at[c, pl.ds(s * rps, rps)])

    if with_gather:
        scratch = [
            pltpu.VMEM((k_chunks, CHUNK), jnp.int32),
            pltpu.VMEM((k_chunks, CHUNK), jnp.int32),
            pltpu.VMEM((NBUF, CHUNK, d), jnp.float32),
            pltpu.VMEM_SHARED((n_pad, d), jnp.float32),
        ] + [pltpu.SemaphoreType.DMA] * NBUF
    else:
        scratch = [
            pltpu.VMEM((k_chunks, CHUNK), jnp.int32),
            pltpu.VMEM((CHUNK, d), jnp.float32),
            pltpu.VMEM_SHARED((n_pad, d), jnp.float32),
        ]

    return pl.kernel(
        body,
        out_type=jax.ShapeDtypeStruct((NC, n_pad, d), jnp.float32),
        mesh=mesh,
        scratch_types=scratch,
        compiler_params=pltpu.CompilerParams(use_tc_tiling_on_sc=False),
    )


def _tc_matmul(x_pad, w):
    def mm(x_ref, w_ref, o_ref):
        o_ref[...] = jnp.dot(x_ref[...], w_ref[...],
                             preferred_element_type=jnp.float32)
    out = jax.ShapeDtypeStruct((x_pad.shape[0], w.shape[1]), jnp.float32)
    return pl.pallas_call(mm, out_shape=out)(x_pad, w)


def _tc_scale1(degp, h1):
    n_pad, d = h1.shape

    def body(degp_ref, h1_ref, dis_ref, h1s_ref):
        deg = degp_ref[0, :, 0:1] + degp_ref[1, :, 0:1] + 1.0
        dis = lax.rsqrt(deg)
        dis_ref[...] = dis
        h1s_ref[...] = h1_ref[...] * dis

    outs = (jax.ShapeDtypeStruct((n_pad, 1), jnp.float32),
            jax.ShapeDtypeStruct((n_pad, d), jnp.float32))
    return pl.pallas_call(body, out_shape=outs)(degp, h1)


def _tc_layer2_prep(p1, h1s, dis, b1r, w2p, n_real):
    n_pad = h1s.shape[0]
    d2 = w2p.shape[1]

    def body(p_ref, h1s_ref, dis_ref, b1_ref, w2_ref, o_ref):
        dis = dis_ref[...]
        agg = (p_ref[0] + p_ref[1] + h1s_ref[...]) * dis + b1_ref[...]
        out1 = jnp.maximum(agg, 0.0)
        h2 = jnp.dot(out1, w2_ref[...], preferred_element_type=jnp.float32)
        mask = lax.broadcasted_iota(jnp.int32, (n_pad, 1), 0) < n_real
        o_ref[...] = jnp.where(mask, h2 * dis, 0.0)

    out = jax.ShapeDtypeStruct((n_pad, d2), jnp.float32)
    return pl.pallas_call(body, out_shape=out)(p1, h1s, dis, b1r, w2p)


def _tc_final(p2, h2s, dis, b2r, n_real, d_out):
    def body(p_ref, h2s_ref, dis_ref, b2_ref, o_ref):
        z = (p_ref[0, :n_real, :d_out] + p_ref[1, :n_real, :d_out]
             + h2s_ref[:n_real, :d_out]) * dis_ref[:n_real] + b2_ref[...]
        m = jnp.max(z, axis=1, keepdims=True)
        zm = z - m
        lse = jnp.log(jnp.sum(jnp.exp(zm), axis=1, keepdims=True))
        o_ref[...] = zm - lse

    out = jax.ShapeDtypeStruct((n_real, d_out), jnp.float32)
    return pl.pallas_call(body, out_shape=out)(p2, h2s, dis, b2r)


def kernel(x, edge_index, W1, b1, W2, b2):
    n, d_in = x.shape
    d_hid = W1.shape[1]
    d_out = W2.shape[1]
    e = edge_index.shape[1]

    n_pad = -(-(n + 1) // (NS * CHUNK)) * (NS * CHUNK)
    d_out_pad = -(-d_out // LANES) * LANES
    kc = -(-e // (NW * CHUNK))
    k_chunks = -(-kc // NBUF) * NBUF
    e_pad = k_chunks * NW * CHUNK

    fill = jnp.full((e_pad - e,), n, jnp.int32)
    src3 = jnp.concatenate([edge_index[0], fill]).reshape(NW, k_chunks, CHUNK)
    dst3 = jnp.concatenate([edge_index[1], fill]).reshape(NW, k_chunks, CHUNK)

    x_pad = jnp.pad(x, ((0, n_pad - n), (0, 0)))
    w2p = jnp.pad(W2, ((0, 0), (0, d_out_pad - d_out)))
    b1r = b1.reshape(1, d_hid)
    b2r = b2.reshape(1, d_out)

    degp = _sc_edge_pass(n_pad, LANES, k_chunks, with_gather=False)(dst3)
    h1 = _tc_matmul(x_pad, W1)
    dis, h1s = _tc_scale1(degp, h1)
    p1 = _sc_edge_pass(n_pad, d_hid, k_chunks, with_gather=True)(h1s, src3, dst3)
    h2s = _tc_layer2_prep(p1, h1s, dis, b1r, w2p, n)
    p2 = _sc_edge_pass(n_pad, d_out_pad, k_chunks, with_gather=True)(h2s, src3, dst3)
    return _tc_final(p2, h2s, dis, b2r, n, d_out)


# speedup vs baseline: 47.5106x; 1.7450x over previous
import jax
import jax.numpy as jnp
from jax import lax
from jax.experimental import pallas as pl
from jax.experimental.pallas import tpu as pltpu
from jax.experimental.pallas import tpu_sc as plsc

NC = 2
NS = 16
NW = NC * NS
LANES = 16
CHUNK = 128
NBUF = 8


def _sc_edge_pass(n_pad, d, k_chunks, with_gather):
    mesh = plsc.VectorSubcoreMesh(core_axis_name="c", subcore_axis_name="s")
    rps = n_pad // NS

    def body(*refs):
        if with_gather:
            (hs_hbm, src_hbm, dst_hbm, out_hbm,
             src_v, dst_v, rows_v, acc, tbl), sems = refs[:9], refs[9:]
        else:
            dst_hbm, out_hbm, dst_v, rows_v, acc = refs
        c = lax.axis_index("c")
        s = lax.axis_index("s")
        wid = s * NC + c
        zbuf = rows_v.at[0] if with_gather else rows_v

        @pl.loop(0, CHUNK)
        def _(i):
            for j in range(d // LANES):
                zbuf.at[i, pl.ds(j * LANES, LANES)][...] = jnp.zeros(
                    (LANES,), jnp.float32)

        @pl.loop(0, rps // CHUNK)
        def _(t):
            pltpu.sync_copy(zbuf, acc.at[pl.ds(s * rps + t * CHUNK, CHUNK)])

        if not with_gather:
            @pl.loop(0, CHUNK)
            def _(i):
                for j in range(d // LANES):
                    rows_v.at[i, pl.ds(j * LANES, LANES)][...] = jnp.full(
                        (LANES,), 1.0, jnp.float32)

        pltpu.sync_copy(dst_hbm.at[wid], dst_v)
        if with_gather:
            pltpu.sync_copy(src_hbm.at[wid], src_v)
            pltpu.sync_copy(hs_hbm.at[pl.ds(s * rps, rps)],
                            tbl.at[pl.ds(s * rps, rps)])
        plsc.subcore_barrier()

        if with_gather:
            for b in range(NBUF):
                pltpu.async_copy(tbl.at[src_v.at[b]], rows_v.at[b], sems[b])

            @pl.loop(0, k_chunks, step=NBUF)
            def _(k):
                for b in range(NBUF):
                    pltpu.make_async_copy(
                        tbl.at[src_v.at[b]], rows_v.at[b], sems[b]).wait()
                    pltpu.sync_copy(rows_v.at[b], acc.at[dst_v.at[k + b]],
                                    add=True)
                    nxt = k + b + NBUF

                    @pl.when(nxt < k_chunks)
                    def _():
                        pltpu.async_copy(
                            tbl.at[src_v.at[nxt]], rows_v.at[b], sems[b])
        else:
            @pl.loop(0, k_chunks)
            def _(k):
                pltpu.sync_copy(rows_v, acc.at[dst_v.at[k]], add=True)

        plsc.subcore_barrier()
        pltpu.sync_copy(acc.at[pl.ds(s * rps, rps)],
                        out_hbm.at[c, pl.ds(s * rps, rps)])

    if with_gather:
        scratch = [
            pltpu.VMEM((k_chunks, CHUNK), jnp.int32),
            pltpu.VMEM((k_chunks, CHUNK), jnp.int32),
            pltpu.VMEM((NBUF, CHUNK, d), jnp.float32),
            pltpu.VMEM_SHARED((n_pad, d), jnp.float32),
            pltpu.VMEM_SHARED((n_pad, d), jnp.float32),
        ] + [pltpu.SemaphoreType.DMA] * NBUF
    else:
        scratch = [
            pltpu.VMEM((k_chunks, CHUNK), jnp.int32),
            pltpu.VMEM((CHUNK, d), jnp.float32),
            pltpu.VMEM_SHARED((n_pad, d), jnp.float32),
        ]

    return pl.kernel(
        body,
        out_type=jax.ShapeDtypeStruct((NC, n_pad, d), jnp.float32),
        mesh=mesh,
        scratch_types=scratch,
        compiler_params=pltpu.CompilerParams(use_tc_tiling_on_sc=False),
    )


def _tc_matmul(x_pad, w):
    def mm(x_ref, w_ref, o_ref):
        o_ref[...] = jnp.dot(x_ref[...], w_ref[...],
                             preferred_element_type=jnp.float32)
    out = jax.ShapeDtypeStruct((x_pad.shape[0], w.shape[1]), jnp.float32)
    return pl.pallas_call(mm, out_shape=out)(x_pad, w)


def _tc_scale1(degp, h1):
    n_pad, d = h1.shape

    def body(degp_ref, h1_ref, dis_ref, h1s_ref):
        deg = degp_ref[0, :, 0:1] + degp_ref[1, :, 0:1] + 1.0
        dis = lax.rsqrt(deg)
        dis_ref[...] = dis
        h1s_ref[...] = h1_ref[...] * dis

    outs = (jax.ShapeDtypeStruct((n_pad, 1), jnp.float32),
            jax.ShapeDtypeStruct((n_pad, d), jnp.float32))
    return pl.pallas_call(body, out_shape=outs)(degp, h1)


def _tc_layer2_prep(p1, h1s, dis, b1r, w2p, n_real):
    n_pad = h1s.shape[0]
    d2 = w2p.shape[1]

    def body(p_ref, h1s_ref, dis_ref, b1_ref, w2_ref, o_ref):
        dis = dis_ref[...]
        agg = (p_ref[0] + p_ref[1] + h1s_ref[...]) * dis + b1_ref[...]
        out1 = jnp.maximum(agg, 0.0)
        h2 = jnp.dot(out1, w2_ref[...], preferred_element_type=jnp.float32)
        mask = lax.broadcasted_iota(jnp.int32, (n_pad, 1), 0) < n_real
        o_ref[...] = jnp.where(mask, h2 * dis, 0.0)

    out = jax.ShapeDtypeStruct((n_pad, d2), jnp.float32)
    return pl.pallas_call(body, out_shape=out)(p1, h1s, dis, b1r, w2p)


def _tc_final(p2, h2s, dis, b2r, n_real, d_out):
    def body(p_ref, h2s_ref, dis_ref, b2_ref, o_ref):
        z = (p_ref[0, :n_real, :d_out] + p_ref[1, :n_real, :d_out]
             + h2s_ref[:n_real, :d_out]) * dis_ref[:n_real] + b2_ref[...]
        m = jnp.max(z, axis=1, keepdims=True)
        zm = z - m
        lse = jnp.log(jnp.sum(jnp.exp(zm), axis=1, keepdims=True))
        o_ref[...] = zm - lse

    out = jax.ShapeDtypeStruct((n_real, d_out), jnp.float32)
    return pl.pallas_call(body, out_shape=out)(p2, h2s, dis, b2r)


def kernel(x, edge_index, W1, b1, W2, b2):
    n, d_in = x.shape
    d_hid = W1.shape[1]
    d_out = W2.shape[1]
    e = edge_index.shape[1]

    n_pad = -(-(n + 1) // (NS * CHUNK)) * (NS * CHUNK)
    d_out_pad = -(-d_out // LANES) * LANES
    kc = -(-e // (NW * CHUNK))
    k_chunks = -(-kc // NBUF) * NBUF
    e_pad = k_chunks * NW * CHUNK

    fill = jnp.full((e_pad - e,), n, jnp.int32)
    src3 = jnp.concatenate([edge_index[0], fill]).reshape(NW, k_chunks, CHUNK)
    dst3 = jnp.concatenate([edge_index[1], fill]).reshape(NW, k_chunks, CHUNK)

    x_pad = jnp.pad(x, ((0, n_pad - n), (0, 0)))
    w2p = jnp.pad(W2, ((0, 0), (0, d_out_pad - d_out)))
    b1r = b1.reshape(1, d_hid)
    b2r = b2.reshape(1, d_out)

    degp = _sc_edge_pass(n_pad, LANES, k_chunks, with_gather=False)(dst3)
    h1 = _tc_matmul(x_pad, W1)
    dis, h1s = _tc_scale1(degp, h1)
    p1 = _sc_edge_pass(n_pad, d_hid, k_chunks, with_gather=True)(h1s, src3, dst3)
    h2s = _tc_layer2_prep(p1, h1s, dis, b1r, w2p, n)
    p2 = _sc_edge_pass(n_pad, d_out_pad, k_chunks, with_gather=True)(h2s, src3, dst3)
    return _tc_final(p2, h2s, dis, b2r, n, d_out)


# speedup vs baseline: 61.0512x; 1.2850x over previous
import jax
import jax.numpy as jnp
from jax import lax
from jax.experimental import pallas as pl
from jax.experimental.pallas import tpu as pltpu
from jax.experimental.pallas import tpu_sc as plsc

NC = 2
NS = 16
NW = NC * NS
LANES = 16
CHUNK = 128
NBUF = 8
SWIN = 16


def _sc_edge_pass(n_pad, d, k_chunks, with_gather):
    mesh = plsc.VectorSubcoreMesh(core_axis_name="c", subcore_axis_name="s")
    rps = n_pad // NS

    def body(*refs):
        if with_gather:
            (hs_hbm, ei_hbm, out_hbm,
             src_v, dst_v, rows_v, acc, tbl), sems = refs[:8], refs[8:]
        else:
            ei_hbm, out_hbm, dst_v, rows_v, acc, sem = refs
        c = lax.axis_index("c")
        s = lax.axis_index("s")
        wid = s * NC + c
        zbuf = rows_v.at[0] if with_gather else rows_v

        @pl.loop(0, CHUNK)
        def _(i):
            for j in range(d // LANES):
                zbuf.at[i, pl.ds(j * LANES, LANES)][...] = jnp.zeros(
                    (LANES,), jnp.float32)

        @pl.loop(0, rps // CHUNK)
        def _(t):
            pltpu.sync_copy(zbuf, acc.at[pl.ds(s * rps + t * CHUNK, CHUNK)])

        if not with_gather:
            @pl.loop(0, CHUNK)
            def _(i):
                for j in range(d // LANES):
                    rows_v.at[i, pl.ds(j * LANES, LANES)][...] = jnp.full(
                        (LANES,), 1.0, jnp.float32)

        pltpu.sync_copy(ei_hbm.at[1, pl.ds(wid * k_chunks, k_chunks)], dst_v)
        if with_gather:
            pltpu.sync_copy(ei_hbm.at[0, pl.ds(wid * k_chunks, k_chunks)],
                            src_v)
            pltpu.sync_copy(hs_hbm.at[pl.ds(s * rps, rps)],
                            tbl.at[pl.ds(s * rps, rps)])
        plsc.subcore_barrier()

        if with_gather:
            for b in range(NBUF):
                pltpu.async_copy(tbl.at[src_v.at[b]], rows_v.at[b], sems[b])

            @pl.loop(0, k_chunks, step=NBUF)
            def _(k):
                for b in range(NBUF):
                    pltpu.make_async_copy(
                        tbl.at[src_v.at[b]], rows_v.at[b], sems[b]).wait()
                    pltpu.sync_copy(rows_v.at[b], acc.at[dst_v.at[k + b]],
                                    add=True)
                    nxt = k + b + NBUF

                    @pl.when(nxt < k_chunks)
                    def _():
                        pltpu.async_copy(
                            tbl.at[src_v.at[nxt]], rows_v.at[b], sems[b])
        else:
            @pl.loop(0, k_chunks)
            def _(k):
                pltpu.async_copy(rows_v, acc.at[dst_v.at[k]], sem, add=True)

                @pl.when(k >= SWIN)
                def _():
                    pltpu.make_async_copy(
                        rows_v, acc.at[dst_v.at[k]], sem).wait()

            @pl.loop(0, SWIN)
            def _(k):
                pltpu.make_async_copy(rows_v, acc.at[dst_v.at[0]], sem).wait()

        plsc.subcore_barrier()
        pltpu.sync_copy(acc.at[pl.ds(s * rps, rps)],
                        out_hbm.at[c, pl.ds(s * rps, rps)])

    if with_gather:
        scratch = [
            pltpu.VMEM((k_chunks, CHUNK), jnp.int32),
            pltpu.VMEM((k_chunks, CHUNK), jnp.int32),
            pltpu.VMEM((NBUF, CHUNK, d), jnp.float32),
            pltpu.VMEM_SHARED((n_pad, d), jnp.float32),
            pltpu.VMEM_SHARED((n_pad, d), jnp.float32),
        ] + [pltpu.SemaphoreType.DMA] * NBUF
    else:
        scratch = [
            pltpu.VMEM((k_chunks, CHUNK), jnp.int32),
            pltpu.VMEM((CHUNK, d), jnp.float32),
            pltpu.VMEM_SHARED((n_pad, d), jnp.float32),
            pltpu.SemaphoreType.DMA,
        ]

    return pl.kernel(
        body,
        out_type=jax.ShapeDtypeStruct((NC, n_pad, d), jnp.float32),
        mesh=mesh,
        scratch_types=scratch,
        compiler_params=pltpu.CompilerParams(use_tc_tiling_on_sc=False),
    )


def _tc_h1_dis(x, w1, degp, n_pad):
    n = x.shape[0]
    d = w1.shape[1]

    def body(x_ref, w_ref, degp_ref, dis_ref, h1s_ref):
        deg = degp_ref[0, :, 0:1] + degp_ref[1, :, 0:1] + 1.0
        dis = lax.rsqrt(deg)
        dis_ref[...] = dis
        h1 = jnp.dot(x_ref[...], w_ref[...],
                     preferred_element_type=jnp.float32)
        h1s_ref[0:n, :] = h1 * dis[0:n]
        h1s_ref[n:n_pad, :] = jnp.zeros((n_pad - n, d), jnp.float32)

    outs = (jax.ShapeDtypeStruct((n_pad, 1), jnp.float32),
            jax.ShapeDtypeStruct((n_pad, d), jnp.float32))
    return pl.pallas_call(body, out_shape=outs)(x, w1, degp)


def _tc_mid(p1, h1s, dis, b1):
    n_pad, d = h1s.shape

    def body(p_ref, h1s_ref, dis_ref, b1_ref, g_ref):
        dis = dis_ref[...]
        agg = (p_ref[0] + p_ref[1] + h1s_ref[...]) * dis + b1_ref[...][None, :]
        g_ref[...] = jnp.maximum(agg, 0.0) * dis

    out = jax.ShapeDtypeStruct((n_pad, d), jnp.float32)
    return pl.pallas_call(body, out_shape=out)(p1, h1s, dis, b1)


def _tc_final(p2, g, dis, w2, b2, n):
    d_out = w2.shape[1]

    def body(p_ref, g_ref, dis_ref, w2_ref, b2_ref, o_ref):
        pre = (p_ref[0, 0:n, :] + p_ref[1, 0:n, :]
               + g_ref[0:n, :]) * dis_ref[0:n]
        z = jnp.dot(pre, w2_ref[...],
                    preferred_element_type=jnp.float32) + b2_ref[...][None, :]
        m = jnp.max(z, axis=1, keepdims=True)
        zm = z - m
        lse = jnp.log(jnp.sum(jnp.exp(zm), axis=1, keepdims=True))
        o_ref[...] = zm - lse

    out = jax.ShapeDtypeStruct((n, d_out), jnp.float32)
    return pl.pallas_call(body, out_shape=out)(p2, g, dis, w2, b2)


def kernel(x, edge_index, W1, b1, W2, b2):
    n, d_in = x.shape
    d_hid = W1.shape[1]
    e = edge_index.shape[1]

    n_pad = -(-(n + 1) // (NS * CHUNK)) * (NS * CHUNK)
    kc = -(-e // (NW * CHUNK))
    k_chunks = -(-kc // NBUF) * NBUF
    e_pad = k_chunks * NW * CHUNK

    ei = jnp.pad(edge_index, ((0, 0), (0, e_pad - e)), constant_values=n)
    ei = ei.reshape(2, NW * k_chunks, CHUNK)

    degp = _sc_edge_pass(n_pad, LANES, k_chunks, with_gather=False)(ei)
    dis, h1s = _tc_h1_dis(x, W1, degp, n_pad)
    p1 = _sc_edge_pass(n_pad, d_hid, k_chunks, with_gather=True)(h1s, ei)
    g = _tc_mid(p1, h1s, dis, b1)
    p2 = _sc_edge_pass(n_pad, d_hid, k_chunks, with_gather=True)(g, ei)
    return _tc_final(p2, g, dis, W2, b2, n)


# speedup vs baseline: 63.5221x; 1.0405x over previous
import jax
import jax.numpy as jnp
from jax import lax
from jax.experimental import pallas as pl
from jax.experimental.pallas import tpu as pltpu
from jax.experimental.pallas import tpu_sc as plsc

NC = 2
NS = 16
NW = NC * NS
LANES = 16
CHUNK = 128
NBUF = 8
NRING = 2 * NBUF
SWIN = 16


def _sc_edge_pass(n_pad, d, k_chunks, with_gather):
    mesh = plsc.VectorSubcoreMesh(core_axis_name="c", subcore_axis_name="s")
    rps = n_pad // NS

    def body(*refs):
        if with_gather:
            (hs_hbm, ei_hbm, out_hbm, src_v, dst_v, rows_v, acc, tbl,
             ld_sem, ld_sem2, ld_sem3), gsems = refs[:11], refs[11:]
        else:
            ei_hbm, out_hbm, dst_v, rows_v, acc, ld_sem, sem = refs
        c = lax.axis_index("c")
        s = lax.axis_index("s")
        wid = s * NC + c
        zbuf = rows_v.at[0] if with_gather else rows_v

        pltpu.async_copy(ei_hbm.at[1, pl.ds(wid * k_chunks, k_chunks)],
                         dst_v, ld_sem)
        if with_gather:
            pltpu.async_copy(ei_hbm.at[0, pl.ds(wid * k_chunks, k_chunks)],
                             src_v, ld_sem2)
            pltpu.async_copy(hs_hbm.at[pl.ds(s * rps, rps)],
                             tbl.at[pl.ds(s * rps, rps)], ld_sem3)

        @pl.loop(0, CHUNK)
        def _(i):
            for j in range(d // LANES):
                zbuf.at[i, pl.ds(j * LANES, LANES)][...] = jnp.zeros(
                    (LANES,), jnp.float32)

        @pl.loop(0, rps // CHUNK)
        def _(t):
            pltpu.sync_copy(zbuf, acc.at[pl.ds(s * rps + t * CHUNK, CHUNK)])

        if not with_gather:
            @pl.loop(0, CHUNK)
            def _(i):
                for j in range(d // LANES):
                    rows_v.at[i, pl.ds(j * LANES, LANES)][...] = jnp.full(
                        (LANES,), 1.0, jnp.float32)

        pltpu.make_async_copy(
            ei_hbm.at[1, pl.ds(wid * k_chunks, k_chunks)], dst_v,
            ld_sem).wait()
        if with_gather:
            pltpu.make_async_copy(
                ei_hbm.at[0, pl.ds(wid * k_chunks, k_chunks)], src_v,
                ld_sem2).wait()
            pltpu.make_async_copy(
                hs_hbm.at[pl.ds(s * rps, rps)], tbl.at[pl.ds(s * rps, rps)],
                ld_sem3).wait()
        plsc.subcore_barrier()

        if with_gather:
            for b in range(NBUF):
                pltpu.async_copy(tbl.at[src_v.at[b]], rows_v.at[b], gsems[b])

            @pl.loop(0, k_chunks, step=NBUF)
            def _(k):
                for b in range(NBUF):
                    pltpu.make_async_copy(
                        tbl.at[src_v.at[b]], rows_v.at[b], gsems[b]).wait()
                    pltpu.sync_copy(rows_v.at[b], acc.at[dst_v.at[k + b]],
                                    add=True)
                    nxt = k + b + NBUF

                    @pl.when(nxt < k_chunks)
                    def _():
                        pltpu.async_copy(
                            tbl.at[src_v.at[nxt]], rows_v.at[b], gsems[b])
        else:
            @pl.loop(0, k_chunks)
            def _(k):
                pltpu.async_copy(rows_v, acc.at[dst_v.at[k]], sem, add=True)

                @pl.when(k >= SWIN)
                def _():
                    pltpu.make_async_copy(
                        rows_v, acc.at[dst_v.at[k]], sem).wait()

            @pl.loop(0, SWIN)
            def _(k):
                pltpu.make_async_copy(rows_v, acc.at[dst_v.at[0]], sem).wait()

        plsc.subcore_barrier()
        pltpu.sync_copy(acc.at[pl.ds(s * rps, rps)],
                        out_hbm.at[c, pl.ds(s * rps, rps)])

    if with_gather:
        scratch = [
            pltpu.VMEM((k_chunks, CHUNK), jnp.int32),
            pltpu.VMEM((k_chunks, CHUNK), jnp.int32),
            pltpu.VMEM((NBUF, CHUNK, d), jnp.float32),
            pltpu.VMEM_SHARED((n_pad, d), jnp.float32),
            pltpu.VMEM_SHARED((n_pad, d), jnp.float32),
            pltpu.SemaphoreType.DMA,
            pltpu.SemaphoreType.DMA,
            pltpu.SemaphoreType.DMA,
        ] + [pltpu.SemaphoreType.DMA] * NBUF
    else:
        scratch = [
            pltpu.VMEM((k_chunks, CHUNK), jnp.int32),
            pltpu.VMEM((CHUNK, d), jnp.float32),
            pltpu.VMEM_SHARED((n_pad, d), jnp.float32),
            pltpu.SemaphoreType.DMA,
            pltpu.SemaphoreType.DMA,
        ]

    return pl.kernel(
        body,
        out_type=jax.ShapeDtypeStruct((NC, n_pad, d), jnp.float32),
        mesh=mesh,
        scratch_types=scratch,
        compiler_params=pltpu.CompilerParams(use_tc_tiling_on_sc=False),
    )


def _tc_matmul(x, w):
    def mm(x_ref, w_ref, o_ref):
        o_ref[...] = jnp.dot(x_ref[...], w_ref[...],
                             preferred_element_type=jnp.float32)
    out = jax.ShapeDtypeStruct((x.shape[0], w.shape[1]), jnp.float32)
    return pl.pallas_call(mm, out_shape=out)(x, w)


def _tc_final(pre, w2, b2):
    n = pre.shape[0]
    d_out = w2.shape[1]

    def body(pre_ref, w2_ref, b2_ref, o_ref):
        z = jnp.dot(pre_ref[...], w2_ref[...],
                    preferred_element_type=jnp.float32) + b2_ref[...][None, :]
        m = jnp.max(z, axis=1, keepdims=True)
        zm = z - m
        lse = jnp.log(jnp.sum(jnp.exp(zm), axis=1, keepdims=True))
        o_ref[...] = zm - lse

    out = jax.ShapeDtypeStruct((n, d_out), jnp.float32)
    return pl.pallas_call(body, out_shape=out)(pre, w2, b2)


def kernel(x, edge_index, W1, b1, W2, b2):
    n, d_in = x.shape
    d_hid = W1.shape[1]
    e = edge_index.shape[1]

    n_pad = -(-(n + 1) // (NS * CHUNK)) * (NS * CHUNK)
    kc = -(-e // (NW * CHUNK))
    k_chunks = -(-kc // NRING) * NRING
    e_pad = k_chunks * NW * CHUNK

    ei = jnp.pad(edge_index, ((0, 0), (0, e_pad - e)), constant_values=n)
    ei = ei.reshape(2, NW * k_chunks, CHUNK)

    degp = _sc_edge_pass(n_pad, LANES, k_chunks, with_gather=False)(ei)
    h1 = _tc_matmul(x, W1)

    deg = degp[0, :, 0:1] + degp[1, :, 0:1] + 1.0
    dis = lax.rsqrt(deg)
    h1s = jnp.pad(h1 * dis[:n], ((0, n_pad - n), (0, 0)))

    p1 = _sc_edge_pass(n_pad, d_hid, k_chunks, with_gather=True)(h1s, ei)
    g = jnp.maximum((p1[0] + p1[1] + h1s) * dis + b1[None, :], 0.0) * dis
    p2 = _sc_edge_pass(n_pad, d_hid, k_chunks, with_gather=True)(g, ei)
    pre = (p2[0, :n] + p2[1, :n] + g[:n]) * dis[:n]
    return _tc_final(pre, W2, b2)
